# Initial kernel scaffold; baseline (speedup 1.0000x reference)
import jax
import jax.numpy as jnp
from jax.experimental import pallas as pl


def kernel(features, type_mask, mp_idx_d0, mp_idx_d1, mp_idx_p0, mp_idx_p1, dst_d0, dst_d1, dst_p0, dst_p1, attn_d0, attn_d1, attn_p0, attn_p1, W_sem_d, b_sem_d, q_sem_d, W_sem_p, b_sem_p, q_sem_p, fc_drug_w, fc_drug_b, fc_protein_w, fc_protein_b):
    raise NotImplementedError("write your pallas kernel here")


# speedup vs baseline: 7.8058x; 7.8058x over previous
import functools

import jax
import jax.numpy as jnp
from jax.experimental import pallas as pl
from jax.experimental.pallas import tpu as pltpu

NT = 4096
D = 128
H = 8
L = 3
HD = H * D
BE = 800
R = 512
BT = 512
NWMAX = NT // R


def _agg_body(w0_ref, nw_ref, g_ref, dst_ref, attn_ref, out_s_ref, out_z_ref,
              *, e_real):
    b = pl.program_id(0)

    @pl.when(b == 0)
    def _():
        out_s_ref[...] = jnp.zeros_like(out_s_ref)
        out_z_ref[...] = jnp.zeros_like(out_z_ref)

    g = g_ref[...]
    h = (g[:, 0, :] + g[:, 1, :] + g[:, 2, :]) * (1.0 / 3.0)
    attn = attn_ref[...]
    scores = jax.lax.dot_general(h, attn, (((1,), (1,)), ((), ())),
                                 preferred_element_type=jnp.float32)
    scores = jnp.where(scores >= 0, scores, 0.2 * scores)
    e = jnp.exp(scores)
    idx = jax.lax.broadcasted_iota(jnp.int32, (BE, 1), 0) + b * BE
    e = jnp.where(idx < e_real, e, 0.0)
    msg = jnp.concatenate([e[:, i:i + 1] * h for i in range(H)], axis=1)
    dstb = dst_ref[...]
    w0 = w0_ref[b]
    nw = nw_ref[b]
    for w in range(NWMAX):
        @pl.when(w < nw)
        def _(w=w):
            ws = (w0 + w) * R
            seg = ws + jax.lax.broadcasted_iota(jnp.int32, (1, R), 1)
            onehot = (dstb == seg).astype(jnp.float32)
            ps = jax.lax.dot_general(onehot, msg, (((0,), (0,)), ((), ())),
                                     preferred_element_type=jnp.float32)
            pz = jax.lax.dot_general(onehot, e, (((0,), (0,)), ((), ())),
                                     preferred_element_type=jnp.float32)
            out_s_ref[pl.ds(ws, R), :] = out_s_ref[pl.ds(ws, R), :] + ps
            out_z_ref[pl.ds(ws, R), :] = out_z_ref[pl.ds(ws, R), :] + pz


def _metapath_agg(gathered, dst, attn):
    e_real = gathered.shape[0]
    pad = (-e_real) % BE
    if pad:
        gathered = jnp.pad(gathered, ((0, pad), (0, 0), (0, 0)))
        dst = jnp.pad(dst, (0, pad), mode="edge")
    epad = e_real + pad
    nb = epad // BE
    starts = jnp.arange(nb, dtype=jnp.int32) * BE
    w0s = dst[starts] // R
    nws = dst[starts + (BE - 1)] // R - w0s + 1
    dst2 = dst.reshape(epad, 1)

    grid_spec = pltpu.PrefetchScalarGridSpec(
        num_scalar_prefetch=2,
        grid=(nb,),
        in_specs=[
            pl.BlockSpec((BE, L, D), lambda b, *_: (b, 0, 0)),
            pl.BlockSpec((BE, 1), lambda b, *_: (b, 0)),
            pl.BlockSpec((H, D), lambda b, *_: (0, 0)),
        ],
        out_specs=[
            pl.BlockSpec((NT, HD), lambda b, *_: (0, 0)),
            pl.BlockSpec((NT, H), lambda b, *_: (0, 0)),
        ],
    )
    out_s, out_z = pl.pallas_call(
        functools.partial(_agg_body, e_real=e_real),
        grid_spec=grid_spec,
        out_shape=[
            jax.ShapeDtypeStruct((NT, HD), jnp.float32),
            jax.ShapeDtypeStruct((NT, H), jnp.float32),
        ],
        compiler_params=pltpu.CompilerParams(
            dimension_semantics=("arbitrary",),
            vmem_limit_bytes=100 * 1024 * 1024,
        ),
    )(w0s, nws, gathered, dst2, attn)
    return out_s, out_z


def _sem_body(s0_ref, z0_ref, s1_ref, z1_ref, w_ref, b_ref, q_ref,
              hm0_ref, hm1_ref, sem_ref):
    i = pl.program_id(0)

    @pl.when(i == 0)
    def _():
        sem_ref[...] = jnp.zeros_like(sem_ref)

    def normalize(s, z):
        cols = [s[:, hh * D:(hh + 1) * D] / (z[:, hh:hh + 1] + 1e-9)
                for hh in range(H)]
        hm = jnp.concatenate(cols, axis=1)
        return jnp.where(hm > 0, hm, jnp.exp(jnp.minimum(hm, 0.0)) - 1.0)

    hm0 = normalize(s0_ref[...], z0_ref[...])
    hm1 = normalize(s1_ref[...], z1_ref[...])
    hm0_ref[...] = hm0
    hm1_ref[...] = hm1
    w = w_ref[...]
    bvec = b_ref[...]
    q = q_ref[...]

    def score(hm):
        t = jnp.tanh(jax.lax.dot_general(hm, w, (((1,), (0,)), ((), ())),
                                         preferred_element_type=jnp.float32)
                     + bvec)
        v = jax.lax.dot_general(t, q, (((1,), (0,)), ((), ())),
                                preferred_element_type=jnp.float32)
        return jnp.sum(v, axis=(0, 1), keepdims=True)

    sem_ref[...] = sem_ref[...] + jnp.concatenate(
        [score(hm0), score(hm1)], axis=1)


def _sem_branch(s0, z0, s1, z1, w_sem, b_sem, q_sem):
    a = w_sem.shape[1]
    hm0, hm1, sems = pl.pallas_call(
        _sem_body,
        grid=(NT // BT,),
        in_specs=[
            pl.BlockSpec((BT, HD), lambda i: (i, 0)),
            pl.BlockSpec((BT, H), lambda i: (i, 0)),
            pl.BlockSpec((BT, HD), lambda i: (i, 0)),
            pl.BlockSpec((BT, H), lambda i: (i, 0)),
            pl.BlockSpec((HD, a), lambda i: (0, 0)),
            pl.BlockSpec((1, a), lambda i: (0, 0)),
            pl.BlockSpec((a, 1), lambda i: (0, 0)),
        ],
        out_specs=[
            pl.BlockSpec((BT, HD), lambda i: (i, 0)),
            pl.BlockSpec((BT, HD), lambda i: (i, 0)),
            pl.BlockSpec((1, 2), lambda i: (0, 0)),
        ],
        out_shape=[
            jax.ShapeDtypeStruct((NT, HD), jnp.float32),
            jax.ShapeDtypeStruct((NT, HD), jnp.float32),
            jax.ShapeDtypeStruct((1, 2), jnp.float32),
        ],
        compiler_params=pltpu.CompilerParams(
            dimension_semantics=("arbitrary",),
        ),
    )(s0, z0, s1, z1, w_sem, b_sem.reshape(1, a), q_sem.reshape(a, 1))
    return hm0, hm1, sems


def _fin_body(hm0_ref, hm1_ref, beta_ref, fw_ref, fb_ref, h_ref, out_ref):
    beta = beta_ref[...]
    h = hm0_ref[...] * beta[0:1, 0:1] + hm1_ref[...] * beta[0:1, 1:2]
    h_ref[...] = h
    logits = jax.lax.dot_general(h, fw_ref[...], (((1,), (1,)), ((), ())),
                                 preferred_element_type=jnp.float32)
    out_ref[...] = logits + fb_ref[...]


def _fin_branch(hm0, hm1, beta, fc_w, fc_b):
    out = fc_w.shape[0]
    h, logits = pl.pallas_call(
        _fin_body,
        grid=(NT // BT,),
        in_specs=[
            pl.BlockSpec((BT, HD), lambda i: (i, 0)),
            pl.BlockSpec((BT, HD), lambda i: (i, 0)),
            pl.BlockSpec((1, 2), lambda i: (0, 0)),
            pl.BlockSpec((out, HD), lambda i: (0, 0)),
            pl.BlockSpec((1, out), lambda i: (0, 0)),
        ],
        out_specs=[
            pl.BlockSpec((BT, HD), lambda i: (i, 0)),
            pl.BlockSpec((BT, out), lambda i: (i, 0)),
        ],
        out_shape=[
            jax.ShapeDtypeStruct((NT, HD), jnp.float32),
            jax.ShapeDtypeStruct((NT, out), jnp.float32),
        ],
        compiler_params=pltpu.CompilerParams(
            dimension_semantics=("arbitrary",),
        ),
    )(hm0, hm1, beta, fc_w, fc_b.reshape(1, out))
    return h, logits


def kernel(features, type_mask, mp_idx_d0, mp_idx_d1, mp_idx_p0, mp_idx_p1,
           dst_d0, dst_d1, dst_p0, dst_p1,
           attn_d0, attn_d1, attn_p0, attn_p1,
           W_sem_d, b_sem_d, q_sem_d, W_sem_p, b_sem_p, q_sem_p,
           fc_drug_w, fc_drug_b, fc_protein_w, fc_protein_b):
    del type_mask

    def agg(mp_idx, dst, attn):
        gathered = jnp.take(features, mp_idx, axis=0)
        return _metapath_agg(gathered, dst, attn)

    s_d0, z_d0 = agg(mp_idx_d0, dst_d0, attn_d0)
    s_d1, z_d1 = agg(mp_idx_d1, dst_d1, attn_d1)
    s_p0, z_p0 = agg(mp_idx_p0, dst_p0, attn_p0)
    s_p1, z_p1 = agg(mp_idx_p1, dst_p1, attn_p1)

    hm_d0, hm_d1, sems_d = _sem_branch(s_d0, z_d0, s_d1, z_d1,
                                       W_sem_d, b_sem_d, q_sem_d)
    hm_p0, hm_p1, sems_p = _sem_branch(s_p0, z_p0, s_p1, z_p1,
                                       W_sem_p, b_sem_p, q_sem_p)

    beta_d = jax.nn.softmax(sems_d / NT, axis=1)
    beta_p = jax.nn.softmax(sems_p / NT, axis=1)

    h_drug, logits_drug = _fin_branch(hm_d0, hm_d1, beta_d,
                                      fc_drug_w, fc_drug_b)
    h_protein, logits_protein = _fin_branch(hm_p0, hm_p1, beta_p,
                                            fc_protein_w, fc_protein_b)
    return (logits_drug, logits_protein, h_drug, h_protein)


# speedup vs baseline: 8.1244x; 1.0408x over previous
import functools

import jax
import jax.numpy as jnp
from jax.experimental import pallas as pl
from jax.experimental.pallas import tpu as pltpu

NT = 4096
D = 128
H = 8
L = 3
HD = H * D
BE = 800
R = 256
BT = 512
NWMAX = NT // R


def _agg_body(w0_ref, nw_ref, g_ref, dst_ref, attn_ref, out_s_ref, out_z_ref,
              *, e_real):
    b = pl.program_id(0)

    @pl.when(b == 0)
    def _():
        out_s_ref[...] = jnp.zeros_like(out_s_ref)
        out_z_ref[...] = jnp.zeros_like(out_z_ref)

    g = g_ref[...]
    h = (g[:, 0, :] + g[:, 1, :] + g[:, 2, :]) * (1.0 / 3.0)
    attn = attn_ref[...]
    scores = jax.lax.dot_general(h, attn, (((1,), (1,)), ((), ())),
                                 preferred_element_type=jnp.float32)
    scores = jnp.where(scores >= 0, scores, 0.2 * scores)
    e = jnp.exp(scores)
    idx = jax.lax.broadcasted_iota(jnp.int32, (BE, 1), 0) + b * BE
    e = jnp.where(idx < e_real, e, 0.0)
    msg = jnp.concatenate([e[:, i:i + 1] * h for i in range(H)], axis=1)
    dstb = dst_ref[...]
    w0 = w0_ref[b]
    nw = nw_ref[b]
    for w in range(NWMAX):
        @pl.when(w < nw)
        def _(w=w):
            ws = (w0 + w) * R
            seg = ws + jax.lax.broadcasted_iota(jnp.int32, (1, R), 1)
            onehot = (dstb == seg).astype(jnp.float32)
            ps = jax.lax.dot_general(onehot, msg, (((0,), (0,)), ((), ())),
                                     preferred_element_type=jnp.float32)
            pz = jax.lax.dot_general(onehot, e, (((0,), (0,)), ((), ())),
                                     preferred_element_type=jnp.float32)
            out_s_ref[pl.ds(ws, R), :] = out_s_ref[pl.ds(ws, R), :] + ps
            out_z_ref[pl.ds(ws, R), :] = out_z_ref[pl.ds(ws, R), :] + pz


def _metapath_agg(gathered, dst, attn):
    e_real = gathered.shape[0]
    pad = (-e_real) % BE
    if pad:
        gathered = jnp.pad(gathered, ((0, pad), (0, 0), (0, 0)))
        dst = jnp.pad(dst, (0, pad), mode="edge")
    epad = e_real + pad
    nb = epad // BE
    starts = jnp.arange(nb, dtype=jnp.int32) * BE
    w0s = dst[starts] // R
    nws = dst[starts + (BE - 1)] // R - w0s + 1
    dst2 = dst.reshape(epad, 1)

    grid_spec = pltpu.PrefetchScalarGridSpec(
        num_scalar_prefetch=2,
        grid=(nb,),
        in_specs=[
            pl.BlockSpec((BE, L, D), lambda b, *_: (b, 0, 0)),
            pl.BlockSpec((BE, 1), lambda b, *_: (b, 0)),
            pl.BlockSpec((H, D), lambda b, *_: (0, 0)),
        ],
        out_specs=[
            pl.BlockSpec((NT, HD), lambda b, *_: (0, 0)),
            pl.BlockSpec((NT, H), lambda b, *_: (0, 0)),
        ],
    )
    out_s, out_z = pl.pallas_call(
        functools.partial(_agg_body, e_real=e_real),
        grid_spec=grid_spec,
        out_shape=[
            jax.ShapeDtypeStruct((NT, HD), jnp.float32),
            jax.ShapeDtypeStruct((NT, H), jnp.float32),
        ],
        compiler_params=pltpu.CompilerParams(
            dimension_semantics=("arbitrary",),
            vmem_limit_bytes=100 * 1024 * 1024,
        ),
    )(w0s, nws, gathered, dst2, attn)
    return out_s, out_z


def _sem_body(s0_ref, z0_ref, s1_ref, z1_ref, w_ref, b_ref, q_ref,
              hm0_ref, hm1_ref, sem_ref):
    i = pl.program_id(0)

    @pl.when(i == 0)
    def _():
        sem_ref[...] = jnp.zeros_like(sem_ref)

    def normalize(s, z):
        cols = [s[:, hh * D:(hh + 1) * D] / (z[:, hh:hh + 1] + 1e-9)
                for hh in range(H)]
        hm = jnp.concatenate(cols, axis=1)
        return jnp.where(hm > 0, hm, jnp.exp(jnp.minimum(hm, 0.0)) - 1.0)

    hm0 = normalize(s0_ref[...], z0_ref[...])
    hm1 = normalize(s1_ref[...], z1_ref[...])
    hm0_ref[...] = hm0
    hm1_ref[...] = hm1
    w = w_ref[...]
    bvec = b_ref[...]
    q = q_ref[...]

    def score(hm):
        t = jnp.tanh(jax.lax.dot_general(hm, w, (((1,), (0,)), ((), ())),
                                         preferred_element_type=jnp.float32)
                     + bvec)
        v = jax.lax.dot_general(t, q, (((1,), (0,)), ((), ())),
                                preferred_element_type=jnp.float32)
        return jnp.sum(v, axis=(0, 1), keepdims=True)

    sem_ref[...] = sem_ref[...] + jnp.concatenate(
        [score(hm0), score(hm1)], axis=1)


def _sem_branch(s0, z0, s1, z1, w_sem, b_sem, q_sem):
    a = w_sem.shape[1]
    hm0, hm1, sems = pl.pallas_call(
        _sem_body,
        grid=(NT // BT,),
        in_specs=[
            pl.BlockSpec((BT, HD), lambda i: (i, 0)),
            pl.BlockSpec((BT, H), lambda i: (i, 0)),
            pl.BlockSpec((BT, HD), lambda i: (i, 0)),
            pl.BlockSpec((BT, H), lambda i: (i, 0)),
            pl.BlockSpec((HD, a), lambda i: (0, 0)),
            pl.BlockSpec((1, a), lambda i: (0, 0)),
            pl.BlockSpec((a, 1), lambda i: (0, 0)),
        ],
        out_specs=[
            pl.BlockSpec((BT, HD), lambda i: (i, 0)),
            pl.BlockSpec((BT, HD), lambda i: (i, 0)),
            pl.BlockSpec((1, 2), lambda i: (0, 0)),
        ],
        out_shape=[
            jax.ShapeDtypeStruct((NT, HD), jnp.float32),
            jax.ShapeDtypeStruct((NT, HD), jnp.float32),
            jax.ShapeDtypeStruct((1, 2), jnp.float32),
        ],
        compiler_params=pltpu.CompilerParams(
            dimension_semantics=("arbitrary",),
        ),
    )(s0, z0, s1, z1, w_sem, b_sem.reshape(1, a), q_sem.reshape(a, 1))
    return hm0, hm1, sems


def _fin_body(hm0_ref, hm1_ref, beta_ref, fw_ref, fb_ref, h_ref, out_ref):
    beta = beta_ref[...]
    h = hm0_ref[...] * beta[0:1, 0:1] + hm1_ref[...] * beta[0:1, 1:2]
    h_ref[...] = h
    logits = jax.lax.dot_general(h, fw_ref[...], (((1,), (1,)), ((), ())),
                                 preferred_element_type=jnp.float32)
    out_ref[...] = logits + fb_ref[...]


def _fin_branch(hm0, hm1, beta, fc_w, fc_b):
    out = fc_w.shape[0]
    h, logits = pl.pallas_call(
        _fin_body,
        grid=(NT // BT,),
        in_specs=[
            pl.BlockSpec((BT, HD), lambda i: (i, 0)),
            pl.BlockSpec((BT, HD), lambda i: (i, 0)),
            pl.BlockSpec((1, 2), lambda i: (0, 0)),
            pl.BlockSpec((out, HD), lambda i: (0, 0)),
            pl.BlockSpec((1, out), lambda i: (0, 0)),
        ],
        out_specs=[
            pl.BlockSpec((BT, HD), lambda i: (i, 0)),
            pl.BlockSpec((BT, out), lambda i: (i, 0)),
        ],
        out_shape=[
            jax.ShapeDtypeStruct((NT, HD), jnp.float32),
            jax.ShapeDtypeStruct((NT, out), jnp.float32),
        ],
        compiler_params=pltpu.CompilerParams(
            dimension_semantics=("arbitrary",),
        ),
    )(hm0, hm1, beta, fc_w, fc_b.reshape(1, out))
    return h, logits


def kernel(features, type_mask, mp_idx_d0, mp_idx_d1, mp_idx_p0, mp_idx_p1,
           dst_d0, dst_d1, dst_p0, dst_p1,
           attn_d0, attn_d1, attn_p0, attn_p1,
           W_sem_d, b_sem_d, q_sem_d, W_sem_p, b_sem_p, q_sem_p,
           fc_drug_w, fc_drug_b, fc_protein_w, fc_protein_b):
    del type_mask

    def agg(mp_idx, dst, attn):
        gathered = jnp.take(features, mp_idx, axis=0)
        return _metapath_agg(gathered, dst, attn)

    s_d0, z_d0 = agg(mp_idx_d0, dst_d0, attn_d0)
    s_d1, z_d1 = agg(mp_idx_d1, dst_d1, attn_d1)
    s_p0, z_p0 = agg(mp_idx_p0, dst_p0, attn_p0)
    s_p1, z_p1 = agg(mp_idx_p1, dst_p1, attn_p1)

    hm_d0, hm_d1, sems_d = _sem_branch(s_d0, z_d0, s_d1, z_d1,
                                       W_sem_d, b_sem_d, q_sem_d)
    hm_p0, hm_p1, sems_p = _sem_branch(s_p0, z_p0, s_p1, z_p1,
                                       W_sem_p, b_sem_p, q_sem_p)

    beta_d = jax.nn.softmax(sems_d / NT, axis=1)
    beta_p = jax.nn.softmax(sems_p / NT, axis=1)

    h_drug, logits_drug = _fin_branch(hm_d0, hm_d1, beta_d,
                                      fc_drug_w, fc_drug_b)
    h_protein, logits_protein = _fin_branch(hm_p0, hm_p1, beta_p,
                                            fc_protein_w, fc_protein_b)
    return (logits_drug, logits_protein, h_drug, h_protein)
